# baseline (speedup 1.0000x reference)
import functools

import jax
import jax.numpy as jnp
from jax.experimental import pallas as pl
from jax.experimental.pallas import tpu as pltpu
from jax.experimental.pallas import tpu_sc as plsc

B, T, DIM, K = 8, 1024, 32, 8192
BKA = 2048
BK = 2048
GW = 128


def _norm_body(sel_ref, w0_ref, w1_ref, nt_ref):
    w0 = w0_ref[...]
    w1 = w1_ref[...]
    s0 = w0 * w0
    s1 = w1 * w1
    acc = jnp.zeros(s0.shape, jnp.float32)
    for b in range(B):
        acc = acc + jnp.where(sel_ref[b] == 1, s1, s0)
    norm = jnp.sqrt(acc)
    ones = jnp.ones((1, BKA), jnp.float32)
    zeros = jnp.zeros((6, BKA), jnp.float32)
    for j, w in ((0, w0), (1, w1)):
        n = w / norm
        e2 = jnp.sum(n * n, axis=0, keepdims=True)
        nt_ref[j] = jnp.concatenate([n, ones, e2, zeros], axis=0)


def _argmin_body(sel_ref, x_ref, nt_ref, g_ref, xt_ref, x2_ref, bv_ref,
                 bi_ref):
    b = pl.program_id(0)
    kt = pl.program_id(1)
    nkt = pl.num_programs(1)

    @pl.when(kt == 0)
    def _():
        bv_ref[...] = jnp.full((T, 128), jnp.inf, jnp.float32)
        bi_ref[...] = jnp.zeros((T, 128), jnp.int32)
        xb = x_ref[0]
        xt_ref[...] = xb * (-2.0)
        x2 = jnp.sum(xb * xb, axis=1, keepdims=True)
        x2_ref[...] = jnp.broadcast_to(x2, (T, 128))

    xs = xt_ref[...]
    x2b = x2_ref[...]
    bv = bv_ref[...]
    bi = bi_ref[...]
    for c in range(BK // 128):
        ntc = nt_ref[0, 0:32, c * 128:(c + 1) * 128]
        s = jax.lax.dot_general(
            xs, ntc, (((1,), (0,)), ((), ())),
            preferred_element_type=jnp.float32)
        e2c = nt_ref[0, 33:34, c * 128:(c + 1) * 128]
        sc = (x2b + s) + e2c
        pred = sc < bv
        bv = jnp.where(pred, sc, bv)
        bi = jnp.where(pred, kt * (BK // 128) + c, bi)
    bv_ref[...] = bv
    bi_ref[...] = bi

    @pl.when(kt == nkt - 1)
    def _():
        lane = jax.lax.broadcasted_iota(jnp.int32, (T, 128), 1)
        kg = bi * 128 + lane
        m = jnp.min(bv, axis=1, keepdims=True)
        cand = jnp.where(bv == m, kg, jnp.int32(2**31 - 1))
        row = jnp.min(cand, axis=1)
        g_ref[0, :, 0] = row + sel_ref[b] * K


def _diff_body(q_ref, x_ref, out_ref, d_ref):
    q = q_ref[:, :DIM]
    out_ref[...] = q
    d = q - x_ref[...]
    d_ref[0, 0] = jnp.sum(d * d) / (B * T * DIM)


def _sc_gather(tables2, gidx):
    mesh = plsc.VectorSubcoreMesh(core_axis_name="core",
                                  subcore_axis_name="subcore")

    @functools.partial(
        pl.kernel,
        out_type=jax.ShapeDtypeStruct((B * T, 128), jnp.float32),
        mesh=mesh)
    def gather_kernel(t_hbm, i_hbm, o_hbm):
        def body(i_vmem, o_vmem):
            pltpu.sync_copy(t_hbm.at[i_vmem.at[0]], o_vmem)

        pltpu.emit_pipeline(
            body,
            grid=(B * T // GW,),
            in_specs=[pl.BlockSpec((1, GW), index_map=lambda i: (0, i))],
            out_specs=[pl.BlockSpec((GW, 128), index_map=lambda i: (i, 0))],
            core_axis_name=("core", "subcore"),
            dimension_semantics=(pltpu.PARALLEL,),
        )(i_hbm, o_hbm)

    return gather_kernel(tables2, gidx)


def kernel(input, input_code_select, W0, W1):
    x = input.astype(jnp.float32)
    sel = input_code_select.astype(jnp.int32)
    w0t = W0.T
    w1t = W1.T

    nt = pl.pallas_call(
        _norm_body,
        grid_spec=pltpu.PrefetchScalarGridSpec(
            num_scalar_prefetch=1,
            grid=(K // BKA,),
            in_specs=[
                pl.BlockSpec((32, BKA), lambda i, sel_ref: (0, i)),
                pl.BlockSpec((32, BKA), lambda i, sel_ref: (0, i)),
            ],
            out_specs=pl.BlockSpec((2, 40, BKA),
                                   lambda i, sel_ref: (0, 0, i)),
        ),
        out_shape=jax.ShapeDtypeStruct((2, 40, K), jnp.float32),
    )(sel, w0t, w1t)

    g = pl.pallas_call(
        _argmin_body,
        grid_spec=pltpu.PrefetchScalarGridSpec(
            num_scalar_prefetch=1,
            grid=(B, K // BK),
            in_specs=[
                pl.BlockSpec((1, T, 32), lambda b, kt, sel_ref: (b, 0, 0)),
                pl.BlockSpec((1, 40, BK),
                             lambda b, kt, sel_ref: (sel_ref[b], 0, kt)),
            ],
            out_specs=pl.BlockSpec((1, T, 1),
                                   lambda b, kt, sel_ref: (b, 0, 0)),
            scratch_shapes=[
                pltpu.VMEM((T, 32), jnp.float32),
                pltpu.VMEM((T, 128), jnp.float32),
                pltpu.VMEM((T, 128), jnp.float32),
                pltpu.VMEM((T, 128), jnp.int32),
            ],
        ),
        out_shape=jax.ShapeDtypeStruct((B, T, 1), jnp.int32),
        compiler_params=pltpu.CompilerParams(
            dimension_semantics=("parallel", "arbitrary")),
    )(sel, x, nt)

    tables2 = jnp.pad(jnp.concatenate([W0, W1], axis=0),
                      ((0, 0), (0, 128 - DIM)))
    gidx = g.reshape(1, B * T)
    qp = _sc_gather(tables2, gidx)

    out32, diff = pl.pallas_call(
        _diff_body,
        in_specs=[
            pl.BlockSpec((B * T, 128), lambda: (0, 0)),
            pl.BlockSpec((B * T, DIM), lambda: (0, 0)),
        ],
        out_specs=[
            pl.BlockSpec((B * T, DIM), lambda: (0, 0)),
            pl.BlockSpec(memory_space=pltpu.SMEM),
        ],
        out_shape=[
            jax.ShapeDtypeStruct((B * T, DIM), jnp.float32),
            jax.ShapeDtypeStruct((1, 1), jnp.float32),
        ],
    )(qp, x.reshape(B * T, DIM))

    out = out32.reshape(B, T, DIM)
    return out, diff[0, 0], input_code_select


# speedup vs baseline: 1.0069x; 1.0069x over previous
import functools

import jax
import jax.numpy as jnp
from jax.experimental import pallas as pl
from jax.experimental.pallas import tpu as pltpu
from jax.experimental.pallas import tpu_sc as plsc

B, T, DIM, K = 8, 1024, 32, 8192
BKA = 2048
BK = 2048
GW = 256


def _norm_body(sel_ref, w0_ref, w1_ref, nt_ref):
    w0 = w0_ref[...]
    w1 = w1_ref[...]
    s0 = w0 * w0
    s1 = w1 * w1
    acc = jnp.zeros(s0.shape, jnp.float32)
    for b in range(B):
        acc = acc + jnp.where(sel_ref[b] == 1, s1, s0)
    norm = jnp.sqrt(acc)
    ones = jnp.ones((1, BKA), jnp.float32)
    zeros = jnp.zeros((6, BKA), jnp.float32)
    for j, w in ((0, w0), (1, w1)):
        n = w / norm
        e2 = jnp.sum(n * n, axis=0, keepdims=True)
        nt_ref[j] = jnp.concatenate([n, ones, e2, zeros], axis=0)


def _argmin_body(sel_ref, x_ref, nt_ref, g_ref, xt_ref, x2_ref, bv_ref,
                 bi_ref):
    b = pl.program_id(0)
    kt = pl.program_id(1)
    nkt = pl.num_programs(1)

    @pl.when(kt == 0)
    def _():
        bv_ref[...] = jnp.full((T, 128), jnp.inf, jnp.float32)
        bi_ref[...] = jnp.zeros((T, 128), jnp.int32)
        xb = x_ref[0]
        xt_ref[...] = xb * (-2.0)
        x2 = jnp.sum(xb * xb, axis=1, keepdims=True)
        x2_ref[...] = jnp.broadcast_to(x2, (T, 128))

    xs = xt_ref[...]
    x2b = x2_ref[...]
    bv = bv_ref[...]
    bi = bi_ref[...]
    for c in range(BK // 128):
        ntc = nt_ref[0, 0:32, c * 128:(c + 1) * 128]
        s = jax.lax.dot_general(
            xs, ntc, (((1,), (0,)), ((), ())),
            preferred_element_type=jnp.float32)
        e2c = nt_ref[0, 33:34, c * 128:(c + 1) * 128]
        sc = (x2b + s) + e2c
        pred = sc < bv
        bv = jnp.where(pred, sc, bv)
        bi = jnp.where(pred, kt * (BK // 128) + c, bi)
    bv_ref[...] = bv
    bi_ref[...] = bi

    @pl.when(kt == nkt - 1)
    def _():
        lane = jax.lax.broadcasted_iota(jnp.int32, (T, 128), 1)
        kg = bi * 128 + lane
        m = jnp.min(bv, axis=1, keepdims=True)
        cand = jnp.where(bv == m, kg, jnp.int32(2**31 - 1))
        row = jnp.min(cand, axis=1)
        g_ref[0, :, 0] = row + sel_ref[b] * K


def _diff_body(q_ref, x_ref, out_ref, d_ref):
    q = q_ref[:, :DIM]
    out_ref[...] = q
    d = q - x_ref[...]
    d_ref[0, 0] = jnp.sum(d * d) / (B * T * DIM)


def _sc_gather(tables2, gidx):
    mesh = plsc.VectorSubcoreMesh(core_axis_name="core",
                                  subcore_axis_name="subcore")

    @functools.partial(
        pl.kernel,
        out_type=jax.ShapeDtypeStruct((B * T, 128), jnp.float32),
        mesh=mesh)
    def gather_kernel(t_hbm, i_hbm, o_hbm):
        def body(i_vmem, o_vmem):
            pltpu.sync_copy(t_hbm.at[i_vmem.at[0]], o_vmem)

        pltpu.emit_pipeline(
            body,
            grid=(B * T // GW,),
            in_specs=[pl.BlockSpec((1, GW), index_map=lambda i: (0, i))],
            out_specs=[pl.BlockSpec((GW, 128), index_map=lambda i: (i, 0))],
            core_axis_name=("core", "subcore"),
            dimension_semantics=(pltpu.PARALLEL,),
        )(i_hbm, o_hbm)

    return gather_kernel(tables2, gidx)


def kernel(input, input_code_select, W0, W1):
    x = input.astype(jnp.float32)
    sel = input_code_select.astype(jnp.int32)
    w0t = W0.T
    w1t = W1.T

    nt = pl.pallas_call(
        _norm_body,
        grid_spec=pltpu.PrefetchScalarGridSpec(
            num_scalar_prefetch=1,
            grid=(K // BKA,),
            in_specs=[
                pl.BlockSpec((32, BKA), lambda i, sel_ref: (0, i)),
                pl.BlockSpec((32, BKA), lambda i, sel_ref: (0, i)),
            ],
            out_specs=pl.BlockSpec((2, 40, BKA),
                                   lambda i, sel_ref: (0, 0, i)),
        ),
        out_shape=jax.ShapeDtypeStruct((2, 40, K), jnp.float32),
    )(sel, w0t, w1t)

    g = pl.pallas_call(
        _argmin_body,
        grid_spec=pltpu.PrefetchScalarGridSpec(
            num_scalar_prefetch=1,
            grid=(B, K // BK),
            in_specs=[
                pl.BlockSpec((1, T, 32), lambda b, kt, sel_ref: (b, 0, 0)),
                pl.BlockSpec((1, 40, BK),
                             lambda b, kt, sel_ref: (sel_ref[b], 0, kt)),
            ],
            out_specs=pl.BlockSpec((1, T, 1),
                                   lambda b, kt, sel_ref: (b, 0, 0)),
            scratch_shapes=[
                pltpu.VMEM((T, 32), jnp.float32),
                pltpu.VMEM((T, 128), jnp.float32),
                pltpu.VMEM((T, 128), jnp.float32),
                pltpu.VMEM((T, 128), jnp.int32),
            ],
        ),
        out_shape=jax.ShapeDtypeStruct((B, T, 1), jnp.int32),
        compiler_params=pltpu.CompilerParams(
            dimension_semantics=("parallel", "arbitrary")),
    )(sel, x, nt)

    tables2 = jnp.pad(jnp.concatenate([W0, W1], axis=0),
                      ((0, 0), (0, 128 - DIM)))
    gidx = g.reshape(1, B * T)
    qp = _sc_gather(tables2, gidx)

    out32, diff = pl.pallas_call(
        _diff_body,
        in_specs=[
            pl.BlockSpec((B * T, 128), lambda: (0, 0)),
            pl.BlockSpec((B * T, DIM), lambda: (0, 0)),
        ],
        out_specs=[
            pl.BlockSpec((B * T, DIM), lambda: (0, 0)),
            pl.BlockSpec(memory_space=pltpu.SMEM),
        ],
        out_shape=[
            jax.ShapeDtypeStruct((B * T, DIM), jnp.float32),
            jax.ShapeDtypeStruct((1, 1), jnp.float32),
        ],
    )(qp, x.reshape(B * T, DIM))

    out = out32.reshape(B, T, DIM)
    return out, diff[0, 0], input_code_select


# speedup vs baseline: 1.0229x; 1.0159x over previous
import functools

import jax
import jax.numpy as jnp
from jax.experimental import pallas as pl
from jax.experimental.pallas import tpu as pltpu
from jax.experimental.pallas import tpu_sc as plsc

B, T, DIM, K = 8, 1024, 32, 8192
BKA = 2048
BK = 4096
GW = 256


def _norm_body(sel_ref, w0_ref, w1_ref, nt_ref):
    w0 = w0_ref[...]
    w1 = w1_ref[...]
    s0 = w0 * w0
    s1 = w1 * w1
    acc = jnp.zeros(s0.shape, jnp.float32)
    for b in range(B):
        acc = acc + jnp.where(sel_ref[b] == 1, s1, s0)
    norm = jnp.sqrt(acc)
    ones = jnp.ones((1, BKA), jnp.float32)
    zeros = jnp.zeros((6, BKA), jnp.float32)
    for j, w in ((0, w0), (1, w1)):
        n = w / norm
        e2 = jnp.sum(n * n, axis=0, keepdims=True)
        nt_ref[j] = jnp.concatenate([n, ones, e2, zeros], axis=0)


def _argmin_body(sel_ref, x_ref, nt_ref, g_ref, xt_ref, x2_ref, bv_ref,
                 bi_ref):
    b = pl.program_id(0)
    kt = pl.program_id(1)
    nkt = pl.num_programs(1)

    @pl.when(kt == 0)
    def _():
        bv_ref[...] = jnp.full((T, 128), jnp.inf, jnp.float32)
        bi_ref[...] = jnp.zeros((T, 128), jnp.int32)
        xb = x_ref[0]
        xt_ref[...] = xb * (-2.0)
        x2 = jnp.sum(xb * xb, axis=1, keepdims=True)
        x2_ref[...] = jnp.broadcast_to(x2, (T, 128))

    xs = xt_ref[...]
    x2b = x2_ref[...]
    bv = bv_ref[...]
    bi = bi_ref[...]
    for c in range(BK // 128):
        ntc = nt_ref[0, 0:32, c * 128:(c + 1) * 128]
        s = jax.lax.dot_general(
            xs, ntc, (((1,), (0,)), ((), ())),
            preferred_element_type=jnp.float32)
        e2c = nt_ref[0, 33:34, c * 128:(c + 1) * 128]
        sc = (x2b + s) + e2c
        pred = sc < bv
        bv = jnp.where(pred, sc, bv)
        bi = jnp.where(pred, kt * (BK // 128) + c, bi)
    bv_ref[...] = bv
    bi_ref[...] = bi

    @pl.when(kt == nkt - 1)
    def _():
        lane = jax.lax.broadcasted_iota(jnp.int32, (T, 128), 1)
        kg = bi * 128 + lane
        m = jnp.min(bv, axis=1, keepdims=True)
        cand = jnp.where(bv == m, kg, jnp.int32(2**31 - 1))
        row = jnp.min(cand, axis=1)
        g_ref[0, :, 0] = row + sel_ref[b] * K


def _diff_body(q_ref, x_ref, out_ref, d_ref):
    q = q_ref[:, :DIM]
    out_ref[...] = q
    d = q - x_ref[...]
    d_ref[0, 0] = jnp.sum(d * d) / (B * T * DIM)


def _sc_gather(tables2, gidx):
    mesh = plsc.VectorSubcoreMesh(core_axis_name="core",
                                  subcore_axis_name="subcore")

    @functools.partial(
        pl.kernel,
        out_type=jax.ShapeDtypeStruct((B * T, 128), jnp.float32),
        mesh=mesh)
    def gather_kernel(t_hbm, i_hbm, o_hbm):
        def body(i_vmem, o_vmem):
            pltpu.sync_copy(t_hbm.at[i_vmem.at[0]], o_vmem)

        pltpu.emit_pipeline(
            body,
            grid=(B * T // GW,),
            in_specs=[pl.BlockSpec((1, GW), index_map=lambda i: (0, i))],
            out_specs=[pl.BlockSpec((GW, 128), index_map=lambda i: (i, 0))],
            core_axis_name=("core", "subcore"),
            dimension_semantics=(pltpu.PARALLEL,),
        )(i_hbm, o_hbm)

    return gather_kernel(tables2, gidx)


def kernel(input, input_code_select, W0, W1):
    x = input.astype(jnp.float32)
    sel = input_code_select.astype(jnp.int32)
    w0t = W0.T
    w1t = W1.T

    nt = pl.pallas_call(
        _norm_body,
        grid_spec=pltpu.PrefetchScalarGridSpec(
            num_scalar_prefetch=1,
            grid=(K // BKA,),
            in_specs=[
                pl.BlockSpec((32, BKA), lambda i, sel_ref: (0, i)),
                pl.BlockSpec((32, BKA), lambda i, sel_ref: (0, i)),
            ],
            out_specs=pl.BlockSpec((2, 40, BKA),
                                   lambda i, sel_ref: (0, 0, i)),
        ),
        out_shape=jax.ShapeDtypeStruct((2, 40, K), jnp.float32),
    )(sel, w0t, w1t)

    g = pl.pallas_call(
        _argmin_body,
        grid_spec=pltpu.PrefetchScalarGridSpec(
            num_scalar_prefetch=1,
            grid=(B, K // BK),
            in_specs=[
                pl.BlockSpec((1, T, 32), lambda b, kt, sel_ref: (b, 0, 0)),
                pl.BlockSpec((1, 40, BK),
                             lambda b, kt, sel_ref: (sel_ref[b], 0, kt)),
            ],
            out_specs=pl.BlockSpec((1, T, 1),
                                   lambda b, kt, sel_ref: (b, 0, 0)),
            scratch_shapes=[
                pltpu.VMEM((T, 32), jnp.float32),
                pltpu.VMEM((T, 128), jnp.float32),
                pltpu.VMEM((T, 128), jnp.float32),
                pltpu.VMEM((T, 128), jnp.int32),
            ],
        ),
        out_shape=jax.ShapeDtypeStruct((B, T, 1), jnp.int32),
        compiler_params=pltpu.CompilerParams(
            dimension_semantics=("parallel", "arbitrary")),
    )(sel, x, nt)

    tables2 = jnp.pad(jnp.concatenate([W0, W1], axis=0),
                      ((0, 0), (0, 128 - DIM)))
    gidx = g.reshape(1, B * T)
    qp = _sc_gather(tables2, gidx)

    out32, diff = pl.pallas_call(
        _diff_body,
        in_specs=[
            pl.BlockSpec((B * T, 128), lambda: (0, 0)),
            pl.BlockSpec((B * T, DIM), lambda: (0, 0)),
        ],
        out_specs=[
            pl.BlockSpec((B * T, DIM), lambda: (0, 0)),
            pl.BlockSpec(memory_space=pltpu.SMEM),
        ],
        out_shape=[
            jax.ShapeDtypeStruct((B * T, DIM), jnp.float32),
            jax.ShapeDtypeStruct((1, 1), jnp.float32),
        ],
    )(qp, x.reshape(B * T, DIM))

    out = out32.reshape(B, T, DIM)
    return out, diff[0, 0], input_code_select


# speedup vs baseline: 1.0354x; 1.0122x over previous
import functools

import jax
import jax.numpy as jnp
from jax.experimental import pallas as pl
from jax.experimental.pallas import tpu as pltpu
from jax.experimental.pallas import tpu_sc as plsc

B, T, DIM, K = 8, 1024, 32, 8192
BKA = 2048
BK = 8192
GW = 256


def _norm_body(sel_ref, w0_ref, w1_ref, nt_ref):
    w0 = w0_ref[...]
    w1 = w1_ref[...]
    s0 = w0 * w0
    s1 = w1 * w1
    acc = jnp.zeros(s0.shape, jnp.float32)
    for b in range(B):
        acc = acc + jnp.where(sel_ref[b] == 1, s1, s0)
    norm = jnp.sqrt(acc)
    ones = jnp.ones((1, BKA), jnp.float32)
    zeros = jnp.zeros((6, BKA), jnp.float32)
    for j, w in ((0, w0), (1, w1)):
        n = w / norm
        e2 = jnp.sum(n * n, axis=0, keepdims=True)
        nt_ref[j] = jnp.concatenate([n, ones, e2, zeros], axis=0)


def _argmin_body(sel_ref, x_ref, nt_ref, g_ref, xt_ref, x2_ref, bv_ref,
                 bi_ref):
    b = pl.program_id(0)
    kt = pl.program_id(1)
    nkt = pl.num_programs(1)

    @pl.when(kt == 0)
    def _():
        bv_ref[...] = jnp.full((T, 128), jnp.inf, jnp.float32)
        bi_ref[...] = jnp.zeros((T, 128), jnp.int32)
        xb = x_ref[0]
        xt_ref[...] = xb * (-2.0)
        x2 = jnp.sum(xb * xb, axis=1, keepdims=True)
        x2_ref[...] = jnp.broadcast_to(x2, (T, 128))

    xs = xt_ref[...]
    x2b = x2_ref[...]
    bv = bv_ref[...]
    bi = bi_ref[...]
    for c in range(BK // 128):
        ntc = nt_ref[0, 0:32, c * 128:(c + 1) * 128]
        s = jax.lax.dot_general(
            xs, ntc, (((1,), (0,)), ((), ())),
            preferred_element_type=jnp.float32)
        e2c = nt_ref[0, 33:34, c * 128:(c + 1) * 128]
        sc = (x2b + s) + e2c
        pred = sc < bv
        bv = jnp.where(pred, sc, bv)
        bi = jnp.where(pred, kt * (BK // 128) + c, bi)
    bv_ref[...] = bv
    bi_ref[...] = bi

    @pl.when(kt == nkt - 1)
    def _():
        lane = jax.lax.broadcasted_iota(jnp.int32, (T, 128), 1)
        kg = bi * 128 + lane
        m = jnp.min(bv, axis=1, keepdims=True)
        cand = jnp.where(bv == m, kg, jnp.int32(2**31 - 1))
        row = jnp.min(cand, axis=1)
        g_ref[0, :, 0] = row + sel_ref[b] * K


def _diff_body(q_ref, x_ref, out_ref, d_ref):
    q = q_ref[:, :DIM]
    out_ref[...] = q
    d = q - x_ref[...]
    d_ref[0, 0] = jnp.sum(d * d) / (B * T * DIM)


def _sc_gather(tables2, gidx):
    mesh = plsc.VectorSubcoreMesh(core_axis_name="core",
                                  subcore_axis_name="subcore")

    @functools.partial(
        pl.kernel,
        out_type=jax.ShapeDtypeStruct((B * T, 128), jnp.float32),
        mesh=mesh)
    def gather_kernel(t_hbm, i_hbm, o_hbm):
        def body(i_vmem, o_vmem):
            pltpu.sync_copy(t_hbm.at[i_vmem.at[0]], o_vmem)

        pltpu.emit_pipeline(
            body,
            grid=(B * T // GW,),
            in_specs=[pl.BlockSpec((1, GW), index_map=lambda i: (0, i))],
            out_specs=[pl.BlockSpec((GW, 128), index_map=lambda i: (i, 0))],
            core_axis_name=("core", "subcore"),
            dimension_semantics=(pltpu.PARALLEL,),
        )(i_hbm, o_hbm)

    return gather_kernel(tables2, gidx)


def kernel(input, input_code_select, W0, W1):
    x = input.astype(jnp.float32)
    sel = input_code_select.astype(jnp.int32)
    w0t = W0.T
    w1t = W1.T

    nt = pl.pallas_call(
        _norm_body,
        grid_spec=pltpu.PrefetchScalarGridSpec(
            num_scalar_prefetch=1,
            grid=(K // BKA,),
            in_specs=[
                pl.BlockSpec((32, BKA), lambda i, sel_ref: (0, i)),
                pl.BlockSpec((32, BKA), lambda i, sel_ref: (0, i)),
            ],
            out_specs=pl.BlockSpec((2, 40, BKA),
                                   lambda i, sel_ref: (0, 0, i)),
        ),
        out_shape=jax.ShapeDtypeStruct((2, 40, K), jnp.float32),
    )(sel, w0t, w1t)

    g = pl.pallas_call(
        _argmin_body,
        grid_spec=pltpu.PrefetchScalarGridSpec(
            num_scalar_prefetch=1,
            grid=(B, K // BK),
            in_specs=[
                pl.BlockSpec((1, T, 32), lambda b, kt, sel_ref: (b, 0, 0)),
                pl.BlockSpec((1, 40, BK),
                             lambda b, kt, sel_ref: (sel_ref[b], 0, kt)),
            ],
            out_specs=pl.BlockSpec((1, T, 1),
                                   lambda b, kt, sel_ref: (b, 0, 0)),
            scratch_shapes=[
                pltpu.VMEM((T, 32), jnp.float32),
                pltpu.VMEM((T, 128), jnp.float32),
                pltpu.VMEM((T, 128), jnp.float32),
                pltpu.VMEM((T, 128), jnp.int32),
            ],
        ),
        out_shape=jax.ShapeDtypeStruct((B, T, 1), jnp.int32),
        compiler_params=pltpu.CompilerParams(
            dimension_semantics=("parallel", "arbitrary")),
    )(sel, x, nt)

    tables2 = jnp.pad(jnp.concatenate([W0, W1], axis=0),
                      ((0, 0), (0, 128 - DIM)))
    gidx = g.reshape(1, B * T)
    qp = _sc_gather(tables2, gidx)

    out32, diff = pl.pallas_call(
        _diff_body,
        in_specs=[
            pl.BlockSpec((B * T, 128), lambda: (0, 0)),
            pl.BlockSpec((B * T, DIM), lambda: (0, 0)),
        ],
        out_specs=[
            pl.BlockSpec((B * T, DIM), lambda: (0, 0)),
            pl.BlockSpec(memory_space=pltpu.SMEM),
        ],
        out_shape=[
            jax.ShapeDtypeStruct((B * T, DIM), jnp.float32),
            jax.ShapeDtypeStruct((1, 1), jnp.float32),
        ],
    )(qp, x.reshape(B * T, DIM))

    out = out32.reshape(B, T, DIM)
    return out, diff[0, 0], input_code_select
